# Initial kernel scaffold; baseline (speedup 1.0000x reference)
import jax
import jax.numpy as jnp
from jax.experimental import pallas as pl


def kernel(video_frames, attn_in_w, attn_in_b, attn_out_w, attn_out_b, ln1_g, ln1_b, ffn_w1, ffn_b1, ffn_w2, ffn_b2, ln2_g, ln2_b):
    raise NotImplementedError("write your pallas kernel here")


# speedup vs baseline: 202.1462x; 202.1462x over previous
import functools

import jax
import jax.numpy as jnp
from jax import lax
from jax.experimental import pallas as pl
from jax.experimental.pallas import tpu as pltpu
from jax.experimental.pallas import tpu_sc as plsc

D = 768
NH = 8
HD = D // NH

_PREC = lax.Precision.DEFAULT


def _top2_rows(sim, n_total):
    cols = lax.broadcasted_iota(jnp.int32, sim.shape, 1)
    big = jnp.int32(n_total + 1)
    m1 = jnp.max(sim, axis=1, keepdims=True)
    c1 = jnp.min(jnp.where(sim == m1, cols, big), axis=1)
    sim2 = jnp.where(cols == c1[:, None], -jnp.inf, sim)
    m2 = jnp.max(sim2, axis=1, keepdims=True)
    c2 = jnp.min(jnp.where(sim2 == m2, cols, big), axis=1)
    return c1, c2


def _token_hashes(t):
    bits = lax.bitcast_convert_type(t, jnp.int32)
    k = lax.broadcasted_iota(jnp.int32, bits.shape, 1)
    a1 = k * 2 + 1
    a2 = (k + 7) * jnp.int32(-1640531527) * 2 + 1
    h1 = jnp.sum(bits * a1, axis=1)
    h2 = jnp.sum(bits * a2 + (bits >> 7), axis=1)
    return h1, h2


def _rep_kernel(hb_ref, h_ref, rep_ref, *, n):
    h1 = h_ref[0, :][None, :]
    h2 = h_ref[1, :][None, :]
    hb1 = hb_ref[0, :][:, None]
    hb2 = hb_ref[1, :][:, None]
    eq = (hb1 == h1) & (hb2 == h2)
    cols = lax.broadcasted_iota(jnp.int32, eq.shape, 1)
    rep = jnp.min(jnp.where(eq, cols, n), axis=1)
    rowid = lax.broadcasted_iota(jnp.int32, rep_ref.shape, 0)
    rep_ref[...] = jnp.where(rowid == 0, rep[None, :], 0)


def _hash_kernel(all_ref, h_ref):
    h1, h2 = _token_hashes(all_ref[...])
    rowid = lax.broadcasted_iota(jnp.int32, h_ref.shape, 0)
    h_ref[...] = jnp.where(rowid == 0, h1[None, :],
                           jnp.where(rowid == 1, h2[None, :], 0))


def _dup_rep(tokens):
    n = tokens.shape[0]
    h = pl.pallas_call(
        _hash_kernel,
        out_shape=jax.ShapeDtypeStruct((8, n), jnp.int32),
    )(tokens)
    cb = 256
    return pl.pallas_call(
        functools.partial(_rep_kernel, n=n),
        grid=((n + cb - 1) // cb,),
        in_specs=[pl.BlockSpec((8, cb), lambda i: (0, i)),
                  pl.BlockSpec((8, n), lambda i: (0, 0))],
        out_specs=pl.BlockSpec((8, cb), lambda i: (0, i)),
        out_shape=jax.ShapeDtypeStruct((8, n), jnp.int32),
    )(h, h)


def _top2_kernel(rows_ref, all_ref, pairs_ref, *, n_total):
    t = all_ref[...]
    nrm = jnp.sqrt(jnp.sum(t * t, axis=1))
    r = rows_ref[...]
    nr = jnp.sqrt(jnp.sum(r * r, axis=1))
    dots = lax.dot_general(r, t, (((1,), (1,)), ((), ())), precision=_PREC)
    den = jnp.maximum(nr[:, None] * nrm[None, :], 1e-8)
    sim = dots / den
    c1, c2 = _top2_rows(sim, n_total)
    rowid = lax.broadcasted_iota(jnp.int32, pairs_ref.shape, 0)
    pairs_ref[...] = jnp.where(rowid == 0, c1[None, :],
                               jnp.where(rowid == 1, c2[None, :], 0))


def _top2_pairs(tokens, m):
    n = tokens.shape[0]
    rb = 128
    grid = (m + rb - 1) // rb
    return pl.pallas_call(
        functools.partial(_top2_kernel, n_total=n),
        grid=(grid,),
        in_specs=[
            pl.BlockSpec((rb, D), lambda i: (i, 0)),
            pl.BlockSpec((n, D), lambda i: (0, 0)),
        ],
        out_specs=pl.BlockSpec((8, rb), lambda i: (0, i)),
        out_shape=jax.ShapeDtypeStruct((8, m), jnp.int32),
    )(tokens, tokens)


def _sim_kernel(rows_ref, all_ref, sim_ref):
    t = all_ref[...]
    nrm = jnp.sqrt(jnp.sum(t * t, axis=1))
    r = rows_ref[...]
    nr = jnp.sqrt(jnp.sum(r * r, axis=1))
    dots = lax.dot_general(r, t, (((1,), (1,)), ((), ())), precision=_PREC)
    den = jnp.maximum(nr[:, None] * nrm[None, :], 1e-8)
    sim_ref[...] = dots / den


def _select_kernel(sim_ref, pairs_ref, *, n_total):
    c1, c2 = _top2_rows(sim_ref[...], n_total)
    rowid = lax.broadcasted_iota(jnp.int32, pairs_ref.shape, 0)
    pairs_ref[...] = jnp.where(rowid == 0, c1[None, :],
                               jnp.where(rowid == 1, c2[None, :], 0))


def _top2_pairs_canon(tokens, m):
    n = tokens.shape[0]
    rep = _dup_rep(tokens)
    rb = 128
    grid = (m + rb - 1) // rb
    sim = pl.pallas_call(
        _sim_kernel,
        grid=(grid,),
        in_specs=[
            pl.BlockSpec((rb, D), lambda i: (i, 0)),
            pl.BlockSpec((n, D), lambda i: (0, 0)),
        ],
        out_specs=pl.BlockSpec((rb, n), lambda i: (i, 0)),
        out_shape=jax.ShapeDtypeStruct((m, n), jnp.float32),
    )(tokens, tokens)
    sim = jnp.take(sim, rep[0, :], axis=1)
    return pl.pallas_call(
        functools.partial(_select_kernel, n_total=n),
        grid=(grid,),
        in_specs=[pl.BlockSpec((rb, n), lambda i: (i, 0))],
        out_specs=pl.BlockSpec((8, rb), lambda i: (0, i)),
        out_shape=jax.ShapeDtypeStruct((8, m), jnp.int32),
    )(sim)


_FPW = 32
_NW = D // _FPW


def _merge_scan(tokens, pairs, keep, wa, wb):
    n = tokens.shape[0]
    m = pairs.shape[1]
    mesh = plsc.VectorSubcoreMesh(core_axis_name="c", subcore_axis_name="s")

    def body(tok_hbm, pairs_hbm, out_hbm, tok_v, pairs_v):
        w = lax.axis_index("s") * 2 + lax.axis_index("c")

        @pl.when(w < _NW)
        def _():
            base = w * _FPW
            pltpu.sync_copy(tok_hbm.at[:, pl.ds(base, _FPW)], tok_v)
            pltpu.sync_copy(pairs_hbm.at[pl.ds(0, 2)],
                            pairs_v.at[:, pl.ds(0, m)])

            def step(i, carry):
                a = pairs_v[0, pl.ds(i, 16)][0]
                b = pairs_v[1, pl.ds(i, 16)][0]
                a0 = tok_v[a, pl.ds(0, 16)]
                a1 = tok_v[a, pl.ds(16, 16)]
                b0 = tok_v[b, pl.ds(0, 16)]
                b1 = tok_v[b, pl.ds(16, 16)]
                n0 = a0 * wa + b0 * wb
                n1 = a1 * wa + b1 * wb
                tok_v[a, pl.ds(0, 16)] = n0
                tok_v[a, pl.ds(16, 16)] = n1
                tok_v[b, pl.ds(0, 16)] = n0
                tok_v[b, pl.ds(16, 16)] = n1
                return carry

            lax.fori_loop(0, m, step, 0)
            pltpu.sync_copy(tok_v.at[pl.ds(0, keep)],
                            out_hbm.at[:, pl.ds(base, _FPW)])

    run = pl.kernel(
        body,
        out_type=jax.ShapeDtypeStruct((keep, D), jnp.float32),
        mesh=mesh,
        compiler_params=pltpu.CompilerParams(use_tc_tiling_on_sc=False),
        scratch_types=[
            pltpu.VMEM((n, _FPW), jnp.float32),
            pltpu.VMEM((2, m + 16), jnp.int32),
        ],
    )
    return run(tokens, pairs)


def _layernorm(x, g, b, eps=1e-5):
    m = x.mean(-1, keepdims=True)
    v = ((x - m) ** 2).mean(-1, keepdims=True)
    return (x - m) / jnp.sqrt(v + eps) * g + b


def _mha_ln_kernel(tok_ref, inw_ref, inb_ref, outw_ref, outb_ref,
                   g_ref, b_ref, t3_ref, *, n):
    x = tok_ref[...]
    qkv = lax.dot_general(x, inw_ref[...], (((1,), (1,)), ((), ())),
                          precision=_PREC) + inb_ref[...]
    scale = 1.0 / (HD ** 0.5)
    heads = []
    for h in range(NH):
        q = qkv[:, h * HD:(h + 1) * HD]
        k = qkv[:, D + h * HD:D + (h + 1) * HD]
        v = qkv[:, 2 * D + h * HD:2 * D + (h + 1) * HD]
        s = lax.dot_general(q, k, (((1,), (1,)), ((), ())),
                            precision=_PREC) * scale
        smax = jnp.max(s, axis=-1, keepdims=True)
        e = jnp.exp(s - smax)
        att = e / jnp.sum(e, axis=-1, keepdims=True)
        heads.append(lax.dot_general(att, v, (((1,), (0,)), ((), ())),
                                     precision=_PREC))
    o = jnp.concatenate(heads, axis=1)
    attn = lax.dot_general(o, outw_ref[...], (((1,), (1,)), ((), ())),
                           precision=_PREC) + outb_ref[...]
    t3_ref[...] = _layernorm(x + attn, g_ref[...], b_ref[...])


def _mha_ln(tokens, inw, inb, outw, outb, g, b):
    n = tokens.shape[0]
    return pl.pallas_call(
        functools.partial(_mha_ln_kernel, n=n),
        out_shape=jax.ShapeDtypeStruct((n, D), jnp.float32),
    )(tokens, inw, inb.reshape(1, -1), outw, outb.reshape(1, -1),
      g.reshape(1, -1), b.reshape(1, -1))


def _ffn_ln_kernel(tok_ref, w1_ref, b1_ref, w2_ref, b2_ref, g_ref, b_ref,
                   out_ref):
    x = tok_ref[...]
    h = lax.dot_general(x, w1_ref[...], (((1,), (1,)), ((), ())),
                        precision=_PREC) + b1_ref[...]
    h = 0.5 * h * (1.0 + lax.erf(h * (2.0 ** -0.5)))
    f = lax.dot_general(h, w2_ref[...], (((1,), (1,)), ((), ())),
                        precision=_PREC) + b2_ref[...]
    out_ref[...] = _layernorm(x + f, g_ref[...], b_ref[...])


def _ffn_ln(tokens, w1, b1, w2, b2, g, b):
    n = tokens.shape[0]
    return pl.pallas_call(
        _ffn_ln_kernel,
        out_shape=jax.ShapeDtypeStruct((n, D), jnp.float32),
    )(tokens, w1, b1.reshape(1, -1), w2, b2.reshape(1, -1),
      g.reshape(1, -1), b.reshape(1, -1))


def kernel(video_frames, attn_in_w, attn_in_b, attn_out_w, attn_out_b,
           ln1_g, ln1_b, ffn_w1, ffn_b1, ffn_w2, ffn_b2, ln2_g, ln2_b):
    tokens = video_frames.reshape(-1, D)

    pairs1 = _top2_pairs(tokens, 1600)
    tokens = _merge_scan(tokens, pairs1, 1600, 0.5, 0.5)

    pairs2 = _top2_pairs_canon(tokens, 800)
    tokens = _merge_scan(tokens, pairs2, 800, 0.5, 0.5)

    tokens = _mha_ln(tokens, attn_in_w, attn_in_b,
                     attn_out_w, attn_out_b, ln1_g, ln1_b)
    pairs3 = _top2_pairs_canon(tokens, 200)
    tokens = _merge_scan(tokens, pairs3, 600, 0.6, 0.4)

    return _ffn_ln(tokens, ffn_w1, ffn_b1, ffn_w2, ffn_b2, ln2_g, ln2_b)
